# baseline (speedup 1.0000x reference)
import jax
import jax.numpy as jnp
from jax import lax
from jax.experimental import pallas as pl
from jax.experimental.pallas import tpu as pltpu

_N = 2048
_D = 768
_NB = 8
_BLK = 512
_AUG = 128
_DA = _D + _AUG
_EPS = 1e-5
_IND4 = 1.0 / (_D ** 0.25)
_D4 = _D ** 0.25
_SHIFT = 64.0 + 1.0 + 28.0
_MU = 28.0


def _body(x_ref, mask_ref, rot_ref, out_ref, xa_s, z_s):
    i = pl.program_id(0)

    @pl.when(i == 0)
    def _prep():
        x = x_ref[...]
        mu = jnp.mean(x, axis=1, keepdims=True)
        var = jnp.mean((x - mu) ** 2, axis=1, keepdims=True)
        xn = (x - mu) * lax.rsqrt(var + _EPS)
        xm = xn * mask_ref[...]
        rot = rot_ref[...]
        s4 = jnp.dot(xn, rot, preferred_element_type=jnp.float32)
        s = jnp.concatenate([s4, -s4], axis=1)
        smax = jnp.max(s, axis=1, keepdims=True)
        idx8 = lax.broadcasted_iota(jnp.int32, s.shape, 1)
        cand = jnp.where(s == smax, idx8, _NB)
        first = jnp.min(cand, axis=1, keepdims=True)
        oh = (idx8 == first).astype(jnp.float32)
        cnt = jnp.sum(oh, axis=0, keepdims=True)
        z_s[...] = float(_N) - lax.dot_general(
            oh, cnt, (((1,), (1,)), ((), ())),
            preferred_element_type=jnp.float32)
        idx128 = lax.broadcasted_iota(jnp.int32, (_N, _AUG), 1)
        aug = (jnp.where(idx128 == first, 8.0, 0.0)
               + jnp.where(idx128 == _NB, 1.0, 0.0))
        xa_s[...] = jnp.concatenate(
            [xm * _IND4, aug], axis=1).astype(jnp.bfloat16)

    @pl.when(i > 0)
    def _attn():
        r0 = (i - 1) * _BLK
        qa = xa_s[pl.ds(r0, _BLK), :]
        xa = xa_s[...]
        s = lax.dot_general(qa, xa, (((1,), (1,)), ((), ())),
                            preferred_element_type=jnp.float32)
        p = jnp.exp(s - _SHIFT)
        acc = lax.dot_general(p.astype(jnp.bfloat16), xa[:, :_D],
                              (((1,), (0,)), ((), ())),
                              preferred_element_type=jnp.float32)
        l = jnp.sum(p, axis=1, keepdims=True)
        den = l + z_s[pl.ds(r0, _BLK), :] * jnp.exp(-_MU)
        out_ref[...] = acc * (_D4 / den)


@jax.jit
def kernel(x, input_mask, rotations):
    x2 = x[0]
    mask2 = input_mask[0][:, None]
    rot = rotations[0, :, -1, :]
    nblk = _N // _BLK
    out = pl.pallas_call(
        _body,
        grid=(nblk + 1,),
        in_specs=[
            pl.BlockSpec((_N, _D), lambda i: (0, 0)),
            pl.BlockSpec((_N, 1), lambda i: (0, 0)),
            pl.BlockSpec((_D, _NB // 2), lambda i: (0, 0)),
        ],
        out_specs=pl.BlockSpec((_BLK, _D),
                               lambda i: (jnp.maximum(i - 1, 0), 0)),
        out_shape=jax.ShapeDtypeStruct((_N, _D), jnp.float32),
        scratch_shapes=[
            pltpu.VMEM((_N, _DA), jnp.bfloat16),
            pltpu.VMEM((_N, 1), jnp.float32),
        ],
    )(x2, mask2, rot)

    return out[None]


# speedup vs baseline: 1.0171x; 1.0171x over previous
import jax
import jax.numpy as jnp
from jax import lax
from jax.experimental import pallas as pl
from jax.experimental.pallas import tpu as pltpu

_N = 2048
_D = 768
_NB = 8
_BLK = 1024
_AUG = 128
_DA = _D + _AUG
_EPS = 1e-5
_IND4 = 1.0 / (_D ** 0.25)
_D4 = _D ** 0.25
_SHIFT = 64.0 + 1.0 + 28.0
_MU = 28.0


def _body(x_ref, mask_ref, rot_ref, out_ref, xa_s, z_s):
    i = pl.program_id(0)

    @pl.when(i == 0)
    def _prep():
        x = x_ref[...]
        mu = jnp.mean(x, axis=1, keepdims=True)
        var = jnp.mean((x - mu) ** 2, axis=1, keepdims=True)
        xn = (x - mu) * lax.rsqrt(var + _EPS)
        xm = xn * mask_ref[...]
        rot = rot_ref[...]
        s4 = jnp.dot(xn, rot, preferred_element_type=jnp.float32)
        s = jnp.concatenate([s4, -s4], axis=1)
        smax = jnp.max(s, axis=1, keepdims=True)
        idx8 = lax.broadcasted_iota(jnp.int32, s.shape, 1)
        cand = jnp.where(s == smax, idx8, _NB)
        first = jnp.min(cand, axis=1, keepdims=True)
        oh = (idx8 == first).astype(jnp.float32)
        cnt = jnp.sum(oh, axis=0, keepdims=True)
        z_s[...] = float(_N) - lax.dot_general(
            oh, cnt, (((1,), (1,)), ((), ())),
            preferred_element_type=jnp.float32)
        idx128 = lax.broadcasted_iota(jnp.int32, (_N, _AUG), 1)
        aug = (jnp.where(idx128 == first, 8.0, 0.0)
               + jnp.where(idx128 == _NB, 1.0, 0.0))
        xa_s[...] = jnp.concatenate(
            [xm * _IND4, aug], axis=1).astype(jnp.bfloat16)

    @pl.when(i > 0)
    def _attn():
        r0 = (i - 1) * _BLK
        qa = xa_s[pl.ds(r0, _BLK), :]
        xa = xa_s[...]
        s = lax.dot_general(qa, xa, (((1,), (1,)), ((), ())),
                            preferred_element_type=jnp.float32)
        p = jnp.exp(s - _SHIFT)
        acc = lax.dot_general(p.astype(jnp.bfloat16), xa[:, :_D],
                              (((1,), (0,)), ((), ())),
                              preferred_element_type=jnp.float32)
        l = jnp.sum(p, axis=1, keepdims=True)
        den = l + z_s[pl.ds(r0, _BLK), :] * jnp.exp(-_MU)
        out_ref[...] = acc * (_D4 / den)


@jax.jit
def kernel(x, input_mask, rotations):
    x2 = x[0]
    mask2 = input_mask[0][:, None]
    rot = rotations[0, :, -1, :]
    nblk = _N // _BLK
    out = pl.pallas_call(
        _body,
        grid=(nblk + 1,),
        in_specs=[
            pl.BlockSpec((_N, _D), lambda i: (0, 0)),
            pl.BlockSpec((_N, 1), lambda i: (0, 0)),
            pl.BlockSpec((_D, _NB // 2), lambda i: (0, 0)),
        ],
        out_specs=pl.BlockSpec((_BLK, _D),
                               lambda i: (jnp.maximum(i - 1, 0), 0)),
        out_shape=jax.ShapeDtypeStruct((_N, _D), jnp.float32),
        scratch_shapes=[
            pltpu.VMEM((_N, _DA), jnp.bfloat16),
            pltpu.VMEM((_N, 1), jnp.float32),
        ],
    )(x2, mask2, rot)

    return out[None]
